# Initial kernel scaffold; baseline (speedup 1.0000x reference)
import jax
import jax.numpy as jnp
from jax.experimental import pallas as pl


def kernel(array):
    raise NotImplementedError("write your pallas kernel here")


# speedup vs baseline: 81.2056x; 81.2056x over previous
import jax
import jax.numpy as jnp
from jax.experimental import pallas as pl
from jax.experimental.pallas import tpu as pltpu

_NB = 31
_LANES = 128
_BLOCK_ROWS = 4096


def _stats_kernel(x_ref, o_ref):
    i = pl.program_id(0)
    x = x_ref[...]
    mn = jnp.min(x)
    mx = jnp.max(x)
    s = jnp.sum(x)
    ss = jnp.sum(x * x)

    @pl.when(i == 0)
    def _init():
        o_ref[0] = mn
        o_ref[1] = mx
        o_ref[2] = s
        o_ref[3] = ss

    @pl.when(i != 0)
    def _acc():
        o_ref[0] = jnp.minimum(o_ref[0], mn)
        o_ref[1] = jnp.maximum(o_ref[1], mx)
        o_ref[2] = o_ref[2] + s
        o_ref[3] = o_ref[3] + ss


def _count_kernel(e_ref, x_ref, o_ref):
    i = pl.program_id(0)
    x = x_ref[...]

    @pl.when(i == 0)
    def _init():
        for j in range(_NB + 1):
            o_ref[j] = 0.0

    for j in range(1, _NB):
        cnt = jnp.sum((x >= e_ref[j]).astype(jnp.float32))
        o_ref[j] = o_ref[j] + cnt


def kernel(array):
    n = array.size
    x2 = array.reshape(-1, _LANES)
    rows = x2.shape[0]
    grid = rows // _BLOCK_ROWS

    stats = pl.pallas_call(
        _stats_kernel,
        grid=(grid,),
        in_specs=[pl.BlockSpec((_BLOCK_ROWS, _LANES), lambda i: (i, 0))],
        out_specs=pl.BlockSpec(memory_space=pltpu.SMEM),
        out_shape=jax.ShapeDtypeStruct((4,), jnp.float32),
    )(x2)
    mn, mx, s, ss = stats[0], stats[1], stats[2], stats[3]
    num = jnp.asarray(n, jnp.int32)

    r0 = jnp.where(mx == mn, mn - 0.5, mn)
    r1 = jnp.where(mx == mn, mx + 0.5, mx)
    edges = jnp.linspace(r0, r1, _NB + 1, dtype=jnp.float32)

    g = pl.pallas_call(
        _count_kernel,
        grid=(grid,),
        in_specs=[
            pl.BlockSpec(memory_space=pltpu.SMEM),
            pl.BlockSpec((_BLOCK_ROWS, _LANES), lambda i: (i, 0)),
        ],
        out_specs=pl.BlockSpec(memory_space=pltpu.SMEM),
        out_shape=jax.ShapeDtypeStruct((_NB + 1,), jnp.float32),
    )(edges, x2)
    gfull = g.at[0].set(jnp.float32(n)).at[_NB].set(0.0)
    counts = gfull[: _NB] - gfull[1 : _NB + 1]
    return (mn, mx, num, s, ss, edges, counts)
